# Initial kernel scaffold; baseline (speedup 1.0000x reference)
import jax
import jax.numpy as jnp
from jax.experimental import pallas as pl


def kernel(x, w_gate, w_noise, W1, b1, W2, b2):
    raise NotImplementedError("write your pallas kernel here")


# speedup vs baseline: 3.3256x; 3.3256x over previous
import functools

import jax
import jax.numpy as jnp
from jax import lax
from jax.experimental import pallas as pl
from jax.experimental.pallas import tpu as pltpu
from jax.experimental.pallas import tpu_sc as plsc

_T, _D, _E, _K = 2048, 768, 8, 2
_H = 4 * _D
_P = _T * _K
_EP = 128
_BLK = 256
_NT = _P // _BLK
_S = _NT + _E - 1
_NW = 32
_SQRT_HALF = 0.7071067811865476


def _gating_body(x_ref, wg_ref, idx_ref, grep_ref, loss_ref):
    x = x_ref[...]
    wg = wg_ref[...]
    logits = jnp.dot(x, wg, preferred_element_type=jnp.float32)
    col = lax.broadcasted_iota(jnp.int32, (_T, _EP), 1)
    valid = col < _E
    neg = jnp.float32(-jnp.inf)
    l = jnp.where(valid, logits, neg)
    m0 = jnp.max(l, axis=1, keepdims=True)
    i0 = jnp.min(jnp.where(l == m0, col, _EP), axis=1, keepdims=True)
    l2 = jnp.where(col == i0, neg, l)
    m1 = jnp.max(l2, axis=1, keepdims=True)
    i1 = jnp.min(jnp.where(l2 == m1, col, _EP), axis=1, keepdims=True)
    d = m1 - m0
    e1 = jnp.exp(d)
    s = 1.0 + e1
    g0 = 1.0 / s
    g1 = e1 / s
    idx_ref[...] = jnp.concatenate([i0, i1], axis=1)
    grep_ref[...] = jnp.concatenate(
        [jnp.broadcast_to(g0, (_T, 16)), jnp.broadcast_to(g1, (_T, 16))], axis=1)
    gd = jnp.where(col == i0, g0, 0.0) + jnp.where(col == i1, g1, 0.0)
    imp = jnp.sum(gd, axis=0, keepdims=True)
    ldc = jnp.sum(jnp.where(gd > 0.0, 1.0, 0.0), axis=0, keepdims=True)
    vmask = (lax.broadcasted_iota(jnp.int32, (1, _EP), 1) < _E).astype(jnp.float32)

    def cv2(v):
        mean = jnp.sum(v * vmask) / _E
        var = jnp.sum(((v - mean) ** 2) * vmask) / (_E - 1)
        return var / (mean * mean + 1e-10)

    loss_ref[...] = jnp.reshape((cv2(imp) + cv2(ldc)) * 1e-2, (1, 1))


def _gating(x, w_gate):
    wg_pad = jnp.zeros((_D, _EP), jnp.float32).at[:, :_E].set(w_gate)
    return pl.pallas_call(
        _gating_body,
        out_shape=(
            jax.ShapeDtypeStruct((_T, 2), jnp.int32),
            jax.ShapeDtypeStruct((_T, 32), jnp.float32),
            jax.ShapeDtypeStruct((1, 1), jnp.float32),
        ),
    )(x, wg_pad)


def _routing(idx):
    ef = idx.reshape(_P)
    oh = (ef[:, None] == jnp.arange(_E, dtype=jnp.int32)[None, :]).astype(jnp.int32)
    csum = jnp.cumsum(oh, axis=0)
    counts = csum[-1]
    off = jnp.concatenate(
        [jnp.zeros((1,), jnp.int32), jnp.cumsum(counts)[:-1].astype(jnp.int32)])
    rank = jnp.take_along_axis(csum, ef[:, None], axis=1)[:, 0] - 1
    pos = (off[ef] + rank).astype(jnp.int32)

    t_ids = (jnp.arange(_NT * _E, dtype=jnp.int32) // _E)
    e_ids = (jnp.arange(_NT * _E, dtype=jnp.int32) % _E)
    g_lo = off[e_ids]
    g_hi = (off + counts)[e_ids]
    st = jnp.maximum(g_lo, t_ids * _BLK)
    en = jnp.minimum(g_hi, (t_ids + 1) * _BLK)
    vld = st < en
    key = jnp.where(vld, t_ids * _E + e_ids, _NT * _E)
    order = jnp.argsort(key)
    nvalid = jnp.sum(vld.astype(jnp.int32))
    sel = order[jnp.minimum(jnp.arange(_S), nvalid - 1)]
    sidx = jnp.arange(_S)
    step_t = t_ids[sel]
    step_e = e_ids[sel]
    step_st = jnp.where(sidx < nvalid, st[sel], 0).astype(jnp.int32)
    step_en = jnp.where(sidx < nvalid, en[sel], 0).astype(jnp.int32)
    step_first = jnp.concatenate(
        [jnp.ones((1,), jnp.int32),
         (step_t[1:] != step_t[:-1]).astype(jnp.int32)])
    return pos, step_t, step_e, step_st, step_en, step_first


def _dispatch(x, pos, tokmap):
    mesh = plsc.VectorSubcoreMesh(core_axis_name="c", subcore_axis_name="s")
    chunk = _P // _NW // 2

    @functools.partial(
        pl.kernel,
        mesh=mesh,
        out_type=jax.ShapeDtypeStruct((_P, _D), jnp.float32),
        scratch_types=[
            pltpu.VMEM((chunk,), jnp.int32),
            pltpu.VMEM((chunk,), jnp.int32),
            pltpu.VMEM((chunk, _D), jnp.float32),
            pltpu.SemaphoreType.DMA,
        ],
    )
    def k(x_hbm, pos_hbm, tok_hbm, out_hbm, tok_v, pos_v, rows_v, sem):
        wid = lax.axis_index("s") * 2 + lax.axis_index("c")

        @pl.loop(0, 2)
        def _half(h):
            base = wid * (_P // _NW) + h * chunk
            pltpu.sync_copy(tok_hbm.at[pl.ds(base, chunk)], tok_v)
            pltpu.sync_copy(pos_hbm.at[pl.ds(base, chunk)], pos_v)
            pltpu.async_copy(x_hbm.at[tok_v], rows_v, sem).wait()
            pltpu.async_copy(rows_v, out_hbm.at[pos_v], sem).wait()

    return k(x, pos, tokmap)


def _gelu(h):
    return 0.5 * h * (1.0 + lax.erf(h * _SQRT_HALF))


def _gmm_body(tref, eref, stref, enref, fref,
              xg_ref, w1_ref, b1_ref, w2_ref, b2_ref, out_ref):
    s = pl.program_id(0)
    row0 = tref[s] * _BLK
    st = stref[s] - row0
    en = enref[s] - row0
    xb = xg_ref[...].astype(jnp.bfloat16)
    w1 = w1_ref[0].astype(jnp.bfloat16)
    h = jnp.dot(xb, w1, preferred_element_type=jnp.float32) + b1_ref[0]
    h = _gelu(h)
    w2 = w2_ref[0].astype(jnp.bfloat16)
    o = jnp.dot(h.astype(jnp.bfloat16), w2,
                preferred_element_type=jnp.float32) + b2_ref[0]
    r = lax.broadcasted_iota(jnp.int32, (_BLK, _D), 0)
    o = jnp.where((r >= st) & (r < en), o, 0.0)

    @pl.when(fref[s] == 1)
    def _init():
        out_ref[...] = o

    @pl.when(fref[s] == 0)
    def _acc():
        out_ref[...] += o


def _gmm(xg, W1, b1, W2, b2, step_t, step_e, step_st, step_en, step_first):
    grid_spec = pltpu.PrefetchScalarGridSpec(
        num_scalar_prefetch=5,
        grid=(_S,),
        in_specs=[
            pl.BlockSpec((_BLK, _D), lambda s, t, e, a, b, f: (t[s], 0)),
            pl.BlockSpec((1, _D, _H), lambda s, t, e, a, b, f: (e[s], 0, 0)),
            pl.BlockSpec((1, 1, _H), lambda s, t, e, a, b, f: (e[s], 0, 0)),
            pl.BlockSpec((1, _H, _D), lambda s, t, e, a, b, f: (e[s], 0, 0)),
            pl.BlockSpec((1, 1, _D), lambda s, t, e, a, b, f: (e[s], 0, 0)),
        ],
        out_specs=pl.BlockSpec((_BLK, _D), lambda s, t, e, a, b, f: (t[s], 0)),
    )
    return pl.pallas_call(
        _gmm_body,
        grid_spec=grid_spec,
        out_shape=jax.ShapeDtypeStruct((_P, _D), jnp.float32),
        compiler_params=pltpu.CompilerParams(
            dimension_semantics=("arbitrary",)),
    )(step_t, step_e, step_st, step_en, step_first, xg, W1,
      b1.reshape(_E, 1, _H), W2, b2.reshape(_E, 1, _D))


def _combine(og, pos, grep):
    mesh = plsc.VectorSubcoreMesh(core_axis_name="c", subcore_axis_name="s")
    ct = _T // _NW // 2

    @functools.partial(
        pl.kernel,
        mesh=mesh,
        out_type=jax.ShapeDtypeStruct((_T, _D), jnp.float32),
        scratch_types=[
            pltpu.VMEM((2 * ct,), jnp.int32),
            pltpu.VMEM((2 * ct, _D), jnp.float32),
            pltpu.VMEM((ct, _D), jnp.float32),
            pltpu.VMEM((ct, 32), jnp.float32),
            pltpu.SemaphoreType.DMA,
        ],
    )
    def k(og_hbm, pos_hbm, grep_hbm, y_hbm, pos_v, rows_v, y_v, g_v, sem):
        wid = lax.axis_index("s") * 2 + lax.axis_index("c")

        @pl.loop(0, 2)
        def _half(h):
            tbase = wid * (_T // _NW) + h * ct
            pbase = 2 * tbase
            pltpu.sync_copy(pos_hbm.at[pl.ds(pbase, 2 * ct)], pos_v)
            pltpu.async_copy(og_hbm.at[pos_v], rows_v, sem).wait()
            pltpu.sync_copy(grep_hbm.at[pl.ds(tbase, ct)], g_v)

            @pl.loop(0, ct)
            def _tok(t):
                g0 = g_v[t, pl.ds(0, 16)]
                g1 = g_v[t, pl.ds(16, 16)]
                for j in range(_D // 16):
                    a = rows_v[2 * t, pl.ds(j * 16, 16)]
                    b = rows_v[2 * t + 1, pl.ds(j * 16, 16)]
                    y_v[t, pl.ds(j * 16, 16)] = g0 * a + g1 * b

            pltpu.sync_copy(y_v, y_hbm.at[pl.ds(tbase, ct)])

    return k(og, pos, grep)


def kernel(x, w_gate, w_noise, W1, b1, W2, b2):
    del w_noise
    idx, grep, loss = _gating(x, w_gate)
    pos, step_t, step_e, step_st, step_en, step_first = _routing(idx)
    tokmap = (jnp.arange(_P, dtype=jnp.int32) // _K)
    xg = _dispatch(x, pos, tokmap)
    og = _gmm(xg, W1, b1, W2, b2, step_t, step_e, step_st, step_en, step_first)
    y = _combine(og, pos, grep)
    return y, jnp.reshape(loss, ())


# speedup vs baseline: 3.6684x; 1.1031x over previous
import functools

import jax
import jax.numpy as jnp
from jax import lax
from jax.experimental import pallas as pl
from jax.experimental.pallas import tpu as pltpu
from jax.experimental.pallas import tpu_sc as plsc

_T, _D, _E, _K = 2048, 768, 8, 2
_H = 4 * _D
_P = _T * _K
_EP = 128
_BLK = 256
_NT = _P // _BLK
_S = _NT + _E - 1
_NW = 32
_SQRT_HALF = 0.7071067811865476


def _gating_body(x_ref, wg_ref, idx_ref, grep_ref, loss_ref,
                 pos_ref, cnt_ref, off_ref):
    x = x_ref[...]
    wg = wg_ref[...]
    logits = jnp.dot(x, wg, preferred_element_type=jnp.float32)
    col = lax.broadcasted_iota(jnp.int32, (_T, _EP), 1)
    valid = col < _E
    neg = jnp.float32(-jnp.inf)
    l = jnp.where(valid, logits, neg)
    m0 = jnp.max(l, axis=1, keepdims=True)
    i0 = jnp.min(jnp.where(l == m0, col, _EP), axis=1, keepdims=True)
    l2 = jnp.where(col == i0, neg, l)
    m1 = jnp.max(l2, axis=1, keepdims=True)
    i1 = jnp.min(jnp.where(l2 == m1, col, _EP), axis=1, keepdims=True)
    d = m1 - m0
    e1 = jnp.exp(d)
    s = 1.0 + e1
    g0 = 1.0 / s
    g1 = e1 / s
    idx_ref[...] = jnp.concatenate([i0, i1], axis=1)
    grep_ref[...] = jnp.concatenate(
        [jnp.broadcast_to(g0, (_T, 16)), jnp.broadcast_to(g1, (_T, 16))], axis=1)
    gd = jnp.where(col == i0, g0, 0.0) + jnp.where(col == i1, g1, 0.0)
    imp = jnp.sum(gd, axis=0, keepdims=True)
    ldc = jnp.sum(jnp.where(gd > 0.0, 1.0, 0.0), axis=0, keepdims=True)
    vmask = (lax.broadcasted_iota(jnp.int32, (1, _EP), 1) < _E).astype(jnp.float32)

    def cv2(v):
        mean = jnp.sum(v * vmask) / _E
        var = jnp.sum(((v - mean) ** 2) * vmask) / (_E - 1)
        return var / (mean * mean + 1e-10)

    loss_ref[...] = jnp.reshape((cv2(imp) + cv2(ldc)) * 1e-2, (1, 1))

    cf = jnp.where(col == i0, 1.0, 0.0) + jnp.where(col == i1, 1.0, 0.0)
    counts = jnp.sum(cf, axis=0, keepdims=True)
    g_, r_ = 16, _T // 16
    ri = lax.broadcasted_iota(jnp.int32, (r_, r_), 0)
    ci = lax.broadcasted_iota(jnp.int32, (r_, r_), 1)
    lr = (ci < ri).astype(jnp.bfloat16)
    c3 = cf.reshape(g_, r_, _EP)
    cume_in = jnp.concatenate(
        [jnp.dot(lr, c3[g].astype(jnp.bfloat16),
                 preferred_element_type=jnp.float32)[None]
         for g in range(g_)], axis=0)
    gs = jnp.sum(c3, axis=1)
    gi = lax.broadcasted_iota(jnp.int32, (g_, g_), 0)
    gj = lax.broadcasted_iota(jnp.int32, (g_, g_), 1)
    lg = (gj < gi).astype(jnp.float32)
    gs_excl = jnp.dot(lg, gs, preferred_element_type=jnp.float32,
                      precision=lax.Precision.HIGHEST)
    cume = (cume_in + gs_excl[:, None, :]).reshape(_T, _EP)
    ui = lax.broadcasted_iota(jnp.int32, (_EP, _EP), 0)
    vi = lax.broadcasted_iota(jnp.int32, (_EP, _EP), 1)
    uf = (ui < vi).astype(jnp.float32)
    off = jnp.dot(counts, uf, preferred_element_type=jnp.float32,
                  precision=lax.Precision.HIGHEST)
    ps = cume + off
    p0 = jnp.sum(jnp.where(col == i0, ps, 0.0), axis=1, keepdims=True)
    p1 = jnp.sum(jnp.where(col == i1, ps, 0.0), axis=1, keepdims=True)
    pos_ref[...] = jnp.concatenate([p0, p1], axis=1).astype(jnp.int32)
    cnt_ref[...] = counts.astype(jnp.int32)
    off_ref[...] = off.astype(jnp.int32)


def _gating(x, w_gate):
    wg_pad = jnp.zeros((_D, _EP), jnp.float32).at[:, :_E].set(w_gate)
    return pl.pallas_call(
        _gating_body,
        out_shape=(
            jax.ShapeDtypeStruct((_T, 2), jnp.int32),
            jax.ShapeDtypeStruct((_T, 32), jnp.float32),
            jax.ShapeDtypeStruct((1, 1), jnp.float32),
            jax.ShapeDtypeStruct((_T, 2), jnp.int32),
            jax.ShapeDtypeStruct((1, _EP), jnp.int32),
            jax.ShapeDtypeStruct((1, _EP), jnp.int32),
        ),
    )(x, wg_pad)


def _step_table(cnt_row, off_row):
    counts = cnt_row[0, :_E]
    off = off_row[0, :_E]
    ft = off // _BLK
    lt = (off + counts - 1) // _BLK
    nt = jnp.where(counts > 0, lt - ft + 1, 0)
    cum_incl = jnp.cumsum(nt)
    base = cum_incl - nt
    nvalid = cum_incl[-1]
    sarr = jnp.arange(_S, dtype=jnp.int32)
    s2 = jnp.minimum(sarr, nvalid - 1)
    step_e = jnp.sum((s2[:, None] >= cum_incl[None, :]).astype(jnp.int32),
                     axis=1)
    step_t = ft[step_e] + (s2 - base[step_e])
    st = jnp.maximum(off[step_e], step_t * _BLK)
    en = jnp.minimum(off[step_e] + counts[step_e], (step_t + 1) * _BLK)
    vld = sarr < nvalid
    step_st = jnp.where(vld, st, 0).astype(jnp.int32)
    step_en = jnp.where(vld, en, 0).astype(jnp.int32)
    step_first = jnp.concatenate(
        [jnp.ones((1,), jnp.int32),
         (step_t[1:] != step_t[:-1]).astype(jnp.int32)])
    return step_t.astype(jnp.int32), step_e.astype(jnp.int32), \
        step_st, step_en, step_first


def _dispatch(x, pos, tokmap):
    mesh = plsc.VectorSubcoreMesh(core_axis_name="c", subcore_axis_name="s")
    chunk = _P // _NW // 2

    @functools.partial(
        pl.kernel,
        mesh=mesh,
        out_type=jax.ShapeDtypeStruct((_P, _D), jnp.float32),
        scratch_types=[
            pltpu.VMEM((chunk,), jnp.int32),
            pltpu.VMEM((chunk,), jnp.int32),
            pltpu.VMEM((chunk, _D), jnp.float32),
            pltpu.SemaphoreType.DMA,
        ],
    )
    def k(x_hbm, pos_hbm, tok_hbm, out_hbm, tok_v, pos_v, rows_v, sem):
        wid = lax.axis_index("s") * 2 + lax.axis_index("c")

        @pl.loop(0, 2)
        def _half(h):
            base = wid * (_P // _NW) + h * chunk
            pltpu.sync_copy(tok_hbm.at[pl.ds(base, chunk)], tok_v)
            pltpu.sync_copy(pos_hbm.at[pl.ds(base, chunk)], pos_v)
            pltpu.async_copy(x_hbm.at[tok_v], rows_v, sem).wait()
            pltpu.async_copy(rows_v, out_hbm.at[pos_v], sem).wait()

    return k(x, pos, tokmap)


def _gelu(h):
    return 0.5 * h * (1.0 + lax.erf(h * _SQRT_HALF))


def _gmm_body(tref, eref, stref, enref, fref,
              xg_ref, w1_ref, b1_ref, w2_ref, b2_ref, out_ref):
    s = pl.program_id(0)
    row0 = tref[s] * _BLK
    st = stref[s] - row0
    en = enref[s] - row0
    xb = xg_ref[...].astype(jnp.bfloat16)
    w1 = w1_ref[0].astype(jnp.bfloat16)
    h = jnp.dot(xb, w1, preferred_element_type=jnp.float32) + b1_ref[0]
    h = _gelu(h)
    w2 = w2_ref[0].astype(jnp.bfloat16)
    o = jnp.dot(h.astype(jnp.bfloat16), w2,
                preferred_element_type=jnp.float32) + b2_ref[0]
    r = lax.broadcasted_iota(jnp.int32, (_BLK, _D), 0)
    o = jnp.where((r >= st) & (r < en), o, 0.0)

    @pl.when(fref[s] == 1)
    def _init():
        out_ref[...] = o

    @pl.when(fref[s] == 0)
    def _acc():
        out_ref[...] += o


def _gmm(xg, W1, b1, W2, b2, step_t, step_e, step_st, step_en, step_first):
    grid_spec = pltpu.PrefetchScalarGridSpec(
        num_scalar_prefetch=5,
        grid=(_S,),
        in_specs=[
            pl.BlockSpec((_BLK, _D), lambda s, t, e, a, b, f: (t[s], 0)),
            pl.BlockSpec((1, _D, _H), lambda s, t, e, a, b, f: (e[s], 0, 0)),
            pl.BlockSpec((1, 1, _H), lambda s, t, e, a, b, f: (e[s], 0, 0)),
            pl.BlockSpec((1, _H, _D), lambda s, t, e, a, b, f: (e[s], 0, 0)),
            pl.BlockSpec((1, 1, _D), lambda s, t, e, a, b, f: (e[s], 0, 0)),
        ],
        out_specs=pl.BlockSpec((_BLK, _D), lambda s, t, e, a, b, f: (t[s], 0)),
    )
    return pl.pallas_call(
        _gmm_body,
        grid_spec=grid_spec,
        out_shape=jax.ShapeDtypeStruct((_P, _D), jnp.float32),
        compiler_params=pltpu.CompilerParams(
            dimension_semantics=("arbitrary",)),
    )(step_t, step_e, step_st, step_en, step_first, xg, W1,
      b1.reshape(_E, 1, _H), W2, b2.reshape(_E, 1, _D))


def _combine(og, pos, grep):
    mesh = plsc.VectorSubcoreMesh(core_axis_name="c", subcore_axis_name="s")
    ct = _T // _NW // 2

    @functools.partial(
        pl.kernel,
        mesh=mesh,
        out_type=jax.ShapeDtypeStruct((_T, _D), jnp.float32),
        scratch_types=[
            pltpu.VMEM((2 * ct,), jnp.int32),
            pltpu.VMEM((2 * ct, _D), jnp.float32),
            pltpu.VMEM((ct, _D), jnp.float32),
            pltpu.VMEM((ct, 32), jnp.float32),
            pltpu.SemaphoreType.DMA,
        ],
    )
    def k(og_hbm, pos_hbm, grep_hbm, y_hbm, pos_v, rows_v, y_v, g_v, sem):
        wid = lax.axis_index("s") * 2 + lax.axis_index("c")

        @pl.loop(0, 2)
        def _half(h):
            tbase = wid * (_T // _NW) + h * ct
            pbase = 2 * tbase
            pltpu.sync_copy(pos_hbm.at[pl.ds(pbase, 2 * ct)], pos_v)
            pltpu.async_copy(og_hbm.at[pos_v], rows_v, sem).wait()
            pltpu.sync_copy(grep_hbm.at[pl.ds(tbase, ct)], g_v)

            @pl.loop(0, ct)
            def _tok(t):
                g0 = g_v[t, pl.ds(0, 16)]
                g1 = g_v[t, pl.ds(16, 16)]
                for j in range(_D // 16):
                    a = rows_v[2 * t, pl.ds(j * 16, 16)]
                    b = rows_v[2 * t + 1, pl.ds(j * 16, 16)]
                    y_v[t, pl.ds(j * 16, 16)] = g0 * a + g1 * b

            pltpu.sync_copy(y_v, y_hbm.at[pl.ds(tbase, ct)])

    return k(og, pos, grep)


def kernel(x, w_gate, w_noise, W1, b1, W2, b2):
    del w_noise
    idx, grep, loss, pos2, cnt_row, off_row = _gating(x, w_gate)
    del idx
    step_t, step_e, step_st, step_en, step_first = _step_table(cnt_row, off_row)
    pos = pos2.reshape(_P)
    tokmap = (jnp.arange(_P, dtype=jnp.int32) // _K)
    xg = _dispatch(x, pos, tokmap)
    og = _gmm(xg, W1, b1, W2, b2, step_t, step_e, step_st, step_en, step_first)
    y = _combine(og, pos, grep)
    return y, jnp.reshape(loss, ())


# speedup vs baseline: 4.1016x; 1.1181x over previous
import functools

import jax
import jax.numpy as jnp
from jax import lax
from jax.experimental import pallas as pl
from jax.experimental.pallas import tpu as pltpu
from jax.experimental.pallas import tpu_sc as plsc

_T, _D, _E, _K = 2048, 768, 8, 2
_H = 4 * _D
_P = _T * _K
_EP = 128
_BLK = 256
_NT = _P // _BLK
_S = _NT + _E - 1
_NW = 32
_SQRT_HALF = 0.7071067811865476


def _gating_body(x_ref, wg_ref, idx_ref, grep_ref, loss_ref,
                 posa_ref, posb_ref, cnt_ref, off_ref):
    x = x_ref[...]
    wg = wg_ref[...]
    logits = jnp.dot(x, wg, preferred_element_type=jnp.float32)
    col = lax.broadcasted_iota(jnp.int32, (_T, _EP), 1)
    valid = col < _E
    neg = jnp.float32(-jnp.inf)
    l = jnp.where(valid, logits, neg)
    m0 = jnp.max(l, axis=1, keepdims=True)
    i0 = jnp.min(jnp.where(l == m0, col, _EP), axis=1, keepdims=True)
    l2 = jnp.where(col == i0, neg, l)
    m1 = jnp.max(l2, axis=1, keepdims=True)
    i1 = jnp.min(jnp.where(l2 == m1, col, _EP), axis=1, keepdims=True)
    d = m1 - m0
    e1 = jnp.exp(d)
    s = 1.0 + e1
    g0 = 1.0 / s
    g1 = e1 / s
    idx_ref[...] = jnp.concatenate([i0, i1], axis=1)
    grep_ref[...] = jnp.concatenate(
        [jnp.broadcast_to(g0, (_T, 16)), jnp.broadcast_to(g1, (_T, 16))], axis=1)
    gd = jnp.where(col == i0, g0, 0.0) + jnp.where(col == i1, g1, 0.0)
    imp = jnp.sum(gd, axis=0, keepdims=True)
    ldc = jnp.sum(jnp.where(gd > 0.0, 1.0, 0.0), axis=0, keepdims=True)
    vmask = (lax.broadcasted_iota(jnp.int32, (1, _EP), 1) < _E).astype(jnp.float32)

    def cv2(v):
        mean = jnp.sum(v * vmask) / _E
        var = jnp.sum(((v - mean) ** 2) * vmask) / (_E - 1)
        return var / (mean * mean + 1e-10)

    loss_ref[...] = jnp.reshape((cv2(imp) + cv2(ldc)) * 1e-2, (1, 1))

    cf = jnp.where(col == i0, 1.0, 0.0) + jnp.where(col == i1, 1.0, 0.0)
    counts = jnp.sum(cf, axis=0, keepdims=True)
    g_, r_ = 16, _T // 16
    ri = lax.broadcasted_iota(jnp.int32, (r_, r_), 0)
    ci = lax.broadcasted_iota(jnp.int32, (r_, r_), 1)
    lr = (ci < ri).astype(jnp.bfloat16)
    c3 = cf.reshape(g_, r_, _EP)
    cume_in = jnp.concatenate(
        [jnp.dot(lr, c3[g].astype(jnp.bfloat16),
                 preferred_element_type=jnp.float32)[None]
         for g in range(g_)], axis=0)
    gs = jnp.sum(c3, axis=1)
    gi = lax.broadcasted_iota(jnp.int32, (g_, g_), 0)
    gj = lax.broadcasted_iota(jnp.int32, (g_, g_), 1)
    lg = (gj < gi).astype(jnp.float32)
    gs_excl = jnp.dot(lg, gs, preferred_element_type=jnp.float32,
                      precision=lax.Precision.HIGHEST)
    cume = (cume_in + gs_excl[:, None, :]).reshape(_T, _EP)
    ui = lax.broadcasted_iota(jnp.int32, (_EP, _EP), 0)
    vi = lax.broadcasted_iota(jnp.int32, (_EP, _EP), 1)
    uf = (ui < vi).astype(jnp.float32)
    off = jnp.dot(counts, uf, preferred_element_type=jnp.float32,
                  precision=lax.Precision.HIGHEST)
    ps = cume + off
    p0 = jnp.sum(jnp.where(col == i0, ps, 0.0), axis=1, keepdims=True)
    p1 = jnp.sum(jnp.where(col == i1, ps, 0.0), axis=1, keepdims=True)
    posa_ref[...] = p0.astype(jnp.int32)
    posb_ref[...] = p1.astype(jnp.int32)
    cnt_ref[...] = counts.astype(jnp.int32)
    off_ref[...] = off.astype(jnp.int32)


def _gating(x, w_gate):
    wg_pad = jnp.zeros((_D, _EP), jnp.float32).at[:, :_E].set(w_gate)
    return pl.pallas_call(
        _gating_body,
        out_shape=(
            jax.ShapeDtypeStruct((_T, 2), jnp.int32),
            jax.ShapeDtypeStruct((_T, 32), jnp.float32),
            jax.ShapeDtypeStruct((1, 1), jnp.float32),
            jax.ShapeDtypeStruct((_T, 1), jnp.int32),
            jax.ShapeDtypeStruct((_T, 1), jnp.int32),
            jax.ShapeDtypeStruct((1, _EP), jnp.int32),
            jax.ShapeDtypeStruct((1, _EP), jnp.int32),
        ),
    )(x, wg_pad)


def _step_table(cnt_row, off_row):
    counts = cnt_row[0, :_E]
    off = off_row[0, :_E]
    ft = off // _BLK
    lt = (off + counts - 1) // _BLK
    nt = jnp.where(counts > 0, lt - ft + 1, 0)
    cum_incl = jnp.cumsum(nt)
    base = cum_incl - nt
    nvalid = cum_incl[-1]
    sarr = jnp.arange(_S, dtype=jnp.int32)
    s2 = jnp.minimum(sarr, nvalid - 1)
    step_e = jnp.sum((s2[:, None] >= cum_incl[None, :]).astype(jnp.int32),
                     axis=1)
    step_t = ft[step_e] + (s2 - base[step_e])
    st = jnp.maximum(off[step_e], step_t * _BLK)
    en = jnp.minimum(off[step_e] + counts[step_e], (step_t + 1) * _BLK)
    vld = sarr < nvalid
    step_st = jnp.where(vld, st, 0).astype(jnp.int32)
    step_en = jnp.where(vld, en, 0).astype(jnp.int32)
    step_first = jnp.concatenate(
        [jnp.ones((1,), jnp.int32),
         (step_t[1:] != step_t[:-1]).astype(jnp.int32)])
    return step_t.astype(jnp.int32), step_e.astype(jnp.int32), \
        step_st, step_en, step_first


def _dispatch(x, posa, posb):
    mesh = plsc.VectorSubcoreMesh(core_axis_name="c", subcore_axis_name="s")
    tw = _T // _NW
    ct = tw // 2

    @functools.partial(
        pl.kernel,
        mesh=mesh,
        out_type=jax.ShapeDtypeStruct((_P, _D), jnp.float32),
        scratch_types=[
            pltpu.VMEM((2, ct, _D), jnp.float32),
            pltpu.VMEM((2, ct), jnp.int32),
            pltpu.VMEM((2, ct), jnp.int32),
            pltpu.SemaphoreType.DMA,
            pltpu.SemaphoreType.DMA,
        ],
    )
    def k(x_hbm, pa_hbm, pb_hbm, out_hbm, xv, pav, pbv, lsem, ssem):
        wid = lax.axis_index("s") * 2 + lax.axis_index("c")
        tb = wid * tw
        cp0 = pltpu.async_copy(x_hbm.at[pl.ds(tb, ct)], xv.at[0], lsem)
        cp1 = pltpu.async_copy(x_hbm.at[pl.ds(tb + ct, ct)], xv.at[1], lsem)
        pltpu.sync_copy(pa_hbm.at[pl.ds(tb, ct)], pav.at[0])
        pltpu.sync_copy(pb_hbm.at[pl.ds(tb, ct)], pbv.at[0])
        pltpu.sync_copy(pa_hbm.at[pl.ds(tb + ct, ct)], pav.at[1])
        pltpu.sync_copy(pb_hbm.at[pl.ds(tb + ct, ct)], pbv.at[1])
        cp0.wait()
        sa0 = pltpu.async_copy(xv.at[0], out_hbm.at[pav.at[0]], ssem)
        sb0 = pltpu.async_copy(xv.at[0], out_hbm.at[pbv.at[0]], ssem)
        cp1.wait()
        sa1 = pltpu.async_copy(xv.at[1], out_hbm.at[pav.at[1]], ssem)
        sb1 = pltpu.async_copy(xv.at[1], out_hbm.at[pbv.at[1]], ssem)
        sa0.wait()
        sb0.wait()
        sa1.wait()
        sb1.wait()

    return k(x, posa, posb)


def _gelu(h):
    return 0.5 * h * (1.0 + lax.erf(h * _SQRT_HALF))


def _gmm_body(tref, eref, stref, enref, fref,
              xg_ref, w1_ref, b1_ref, w2_ref, b2_ref, out_ref):
    s = pl.program_id(0)
    row0 = tref[s] * _BLK
    st = stref[s] - row0
    en = enref[s] - row0
    xb = xg_ref[...].astype(jnp.bfloat16)
    w1 = w1_ref[0].astype(jnp.bfloat16)
    h = jnp.dot(xb, w1, preferred_element_type=jnp.float32) + b1_ref[0]
    h = _gelu(h)
    w2 = w2_ref[0].astype(jnp.bfloat16)
    o = jnp.dot(h.astype(jnp.bfloat16), w2,
                preferred_element_type=jnp.float32) + b2_ref[0]
    r = lax.broadcasted_iota(jnp.int32, (_BLK, _D), 0)
    o = jnp.where((r >= st) & (r < en), o, 0.0)

    @pl.when(fref[s] == 1)
    def _init():
        out_ref[...] = o

    @pl.when(fref[s] == 0)
    def _acc():
        out_ref[...] += o


def _gmm(xg, W1, b1, W2, b2, step_t, step_e, step_st, step_en, step_first):
    grid_spec = pltpu.PrefetchScalarGridSpec(
        num_scalar_prefetch=5,
        grid=(_S,),
        in_specs=[
            pl.BlockSpec((_BLK, _D), lambda s, t, e, a, b, f: (t[s], 0)),
            pl.BlockSpec((1, _D, _H), lambda s, t, e, a, b, f: (e[s], 0, 0)),
            pl.BlockSpec((1, 1, _H), lambda s, t, e, a, b, f: (e[s], 0, 0)),
            pl.BlockSpec((1, _H, _D), lambda s, t, e, a, b, f: (e[s], 0, 0)),
            pl.BlockSpec((1, 1, _D), lambda s, t, e, a, b, f: (e[s], 0, 0)),
        ],
        out_specs=pl.BlockSpec((_BLK, _D), lambda s, t, e, a, b, f: (t[s], 0)),
    )
    return pl.pallas_call(
        _gmm_body,
        grid_spec=grid_spec,
        out_shape=jax.ShapeDtypeStruct((_P, _D), jnp.float32),
        compiler_params=pltpu.CompilerParams(
            dimension_semantics=("arbitrary",)),
    )(step_t, step_e, step_st, step_en, step_first, xg, W1,
      b1.reshape(_E, 1, _H), W2, b2.reshape(_E, 1, _D))


def _combine(og, posa, posb, grep):
    mesh = plsc.VectorSubcoreMesh(core_axis_name="c", subcore_axis_name="s")
    ct = _T // _NW // 2

    @functools.partial(
        pl.kernel,
        mesh=mesh,
        out_type=jax.ShapeDtypeStruct((_T, _D), jnp.float32),
        scratch_types=[
            pltpu.VMEM((ct,), jnp.int32),
            pltpu.VMEM((ct,), jnp.int32),
            pltpu.VMEM((ct, _D), jnp.float32),
            pltpu.VMEM((ct, _D), jnp.float32),
            pltpu.VMEM((ct, _D), jnp.float32),
            pltpu.VMEM((ct, 32), jnp.float32),
            pltpu.SemaphoreType.DMA,
        ],
    )
    def k(og_hbm, pa_hbm, pb_hbm, grep_hbm, y_hbm,
          pa_v, pb_v, rows_a, rows_b, y_v, g_v, sem):
        wid = lax.axis_index("s") * 2 + lax.axis_index("c")

        @pl.loop(0, 2)
        def _half(h):
            tbase = wid * (_T // _NW) + h * ct
            pltpu.sync_copy(pa_hbm.at[pl.ds(tbase, ct)], pa_v)
            pltpu.sync_copy(pb_hbm.at[pl.ds(tbase, ct)], pb_v)
            ca = pltpu.async_copy(og_hbm.at[pa_v], rows_a, sem)
            cb = pltpu.async_copy(og_hbm.at[pb_v], rows_b, sem)
            pltpu.sync_copy(grep_hbm.at[pl.ds(tbase, ct)], g_v)
            ca.wait()
            cb.wait()

            @pl.loop(0, ct)
            def _tok(t):
                g0 = g_v[t, pl.ds(0, 16)]
                g1 = g_v[t, pl.ds(16, 16)]
                for j in range(_D // 16):
                    a = rows_a[t, pl.ds(j * 16, 16)]
                    b = rows_b[t, pl.ds(j * 16, 16)]
                    y_v[t, pl.ds(j * 16, 16)] = g0 * a + g1 * b

            pltpu.sync_copy(y_v, y_hbm.at[pl.ds(tbase, ct)])

    return k(og, posa, posb, grep)


def kernel(x, w_gate, w_noise, W1, b1, W2, b2):
    del w_noise
    idx, grep, loss, posa2, posb2, cnt_row, off_row = _gating(x, w_gate)
    del idx
    step_t, step_e, step_st, step_en, step_first = _step_table(cnt_row, off_row)
    posa = posa2.reshape(_T)
    posb = posb2.reshape(_T)
    xg = _dispatch(x, posa, posb)
    og = _gmm(xg, W1, b1, W2, b2, step_t, step_e, step_st, step_en, step_first)
    y = _combine(og, posa, posb, grep)
    return y, jnp.reshape(loss, ())
